# baseline (speedup 1.0000x reference)
import functools

import jax
import jax.numpy as jnp
from jax import lax
from jax.experimental import pallas as pl
from jax.experimental.pallas import tpu as pltpu
from jax.experimental.pallas import tpu_sc as plsc

B = 16384
D = 32
NC = 2
NS = 16
NW = NC * NS
BPW = B // NW
CHUNK = 128
NCHUNK = BPW // CHUNK


@functools.lru_cache(maxsize=None)
def _make_sc_gather():
    mesh = plsc.VectorSubcoreMesh(
        core_axis_name="c", subcore_axis_name="s", num_cores=NC, num_subcores=NS
    )

    @functools.partial(
        pl.kernel,
        out_type=jax.ShapeDtypeStruct((4, B, D), jnp.float32),
        mesh=mesh,
        scratch_types=[
            pltpu.VMEM((NCHUNK, CHUNK), jnp.int32),
            pltpu.VMEM((NCHUNK, CHUNK), jnp.int32),
            pltpu.VMEM((BPW, D), jnp.float32),
            pltpu.VMEM((BPW, D), jnp.float32),
            pltpu.VMEM((BPW, D), jnp.float32),
            pltpu.VMEM((BPW, D), jnp.float32),
            pltpu.SemaphoreType.DMA,
        ],
        compiler_params=pltpu.CompilerParams(
            use_tc_tiling_on_sc=False,
            needs_layout_passes=True,
            skip_device_barrier=True,
        ),
    )
    def _sc_gather(uid_h, jid_h, umt_h, jmt_h, ugt_h, jgt_h, out,
                   uidx, jidx, bum, bjm, bug, bjg, sem):
        wid = lax.axis_index("s") * NC + lax.axis_index("c")
        r0 = wid * NCHUNK
        pltpu.sync_copy(uid_h.at[pl.ds(r0, NCHUNK)], uidx)
        pltpu.sync_copy(jid_h.at[pl.ds(r0, NCHUNK)], jidx)
        copies = []
        for c in range(NCHUNK):
            dst = pl.ds(c * CHUNK, CHUNK)
            copies.append(pltpu.async_copy(umt_h.at[uidx.at[c]], bum.at[dst], sem))
            copies.append(pltpu.async_copy(jmt_h.at[jidx.at[c]], bjm.at[dst], sem))
            copies.append(pltpu.async_copy(ugt_h.at[uidx.at[c]], bug.at[dst], sem))
            copies.append(pltpu.async_copy(jgt_h.at[jidx.at[c]], bjg.at[dst], sem))
        for cp in copies:
            cp.wait()
        base = wid * BPW
        pltpu.sync_copy(bum, out.at[0, pl.ds(base, BPW)])
        pltpu.sync_copy(bjm, out.at[1, pl.ds(base, BPW)])
        pltpu.sync_copy(bug, out.at[2, pl.ds(base, BPW)])
        pltpu.sync_copy(bjg, out.at[3, pl.ds(base, BPW)])

    return _sc_gather


BLK = 2048


def _tc_body(g, w1, b1, w2, b2, w3, b3, w4, w5, b4, b5, out):
    um = g[0]
    jm = g[1]
    ug = g[2]
    jg = g[3]
    w1v = w1[:]
    x = jnp.maximum(um @ w1v[:D, :] + jm @ w1v[D:, :] + b1[:], 0.0)
    x = jnp.maximum(x @ w2[:] + b2[:], 0.0)
    x = jnp.maximum(x @ w3[:] + b3[:], 0.0)
    x = jnp.maximum(x @ w4[:] + b4[0], 0.0)
    dot = jnp.sum(ug * jg, axis=1, keepdims=True)
    su = jnp.sum(ug * ug, axis=1, keepdims=True)
    sj = jnp.sum(jg * jg, axis=1, keepdims=True)
    gmf = dot * lax.rsqrt(jnp.maximum(su, 1e-12)) * lax.rsqrt(jnp.maximum(sj, 1e-12))
    out[:] = x * w5[0, 0] + gmf * w5[1, 0] + b5[0]


def _tc_dense(g, w1, b1, w2, b2, w3, b3, w4, w5, b4, b5):
    full = lambda a: pl.BlockSpec(a.shape, lambda i, _n=a.ndim: (0,) * _n)
    smem = pl.BlockSpec(memory_space=pltpu.SMEM)
    return pl.pallas_call(
        _tc_body,
        grid=(B // BLK,),
        in_specs=[pl.BlockSpec((4, BLK, D), lambda i: (0, i, 0)),
                  full(w1), full(b1), full(w2), full(b2), full(w3), full(b3),
                  full(w4), smem, smem, smem],
        out_specs=pl.BlockSpec((BLK, 1), lambda i: (i, 0)),
        out_shape=jax.ShapeDtypeStruct((B, 1), jnp.float32),
    )(g, w1, b1, w2, b2, w3, b3, w4, w5, b4, b5)


def kernel(user_ids, joke_ids, user_mlp_table, joke_mlp_table,
           user_gmf_table, joke_gmf_table,
           W1, b1, W2, b2, W3, b3, W4, b4, W5, b5):
    uid = user_ids.astype(jnp.int32).reshape(B // CHUNK, CHUNK)
    jid = joke_ids.astype(jnp.int32).reshape(B // CHUNK, CHUNK)
    g = _make_sc_gather()(uid, jid, user_mlp_table, joke_mlp_table,
                          user_gmf_table, joke_gmf_table)
    return _tc_dense(g, W1, b1, W2, b2, W3, b3, W4, W5, b4, b5)
